# baseline (speedup 1.0000x reference)
import functools

import jax
import jax.numpy as jnp
from jax import lax
from jax.experimental import pallas as pl
from jax.experimental.pallas import tpu as pltpu
from jax.experimental.pallas import tpu_sc as plsc

NC = 2
NS = 16
LANES = 16
CH = 128


def _sc_mesh():
    return plsc.VectorSubcoreMesh(
        core_axis_name="c", subcore_axis_name="s", num_cores=NC, num_subcores=NS)


def _fill_zero(ref, nrows, ncols):
    z = jnp.zeros((LANES,), jnp.float32)
    for r in range(nrows):
        for j in range(ncols // LANES):
            ref[r, pl.ds(j * LANES, LANES)] = z


def _make_segsum(n, h, npad, ept):
    rpt = npad // NS
    nch = ept // CH
    assert rpt % 64 == 0 and nch * CH == ept

    @functools.partial(
        pl.kernel,
        out_type=jax.ShapeDtypeStruct((2 * npad, h), jnp.float32),
        mesh=_sc_mesh(),
        scratch_types=(
            pltpu.VMEM_SHARED((npad, h), jnp.float32),
            pltpu.VMEM((CH,), jnp.int32),
            pltpu.VMEM((CH,), jnp.int32),
            pltpu.VMEM((CH, h), jnp.float32),
            pltpu.VMEM((64, h), jnp.float32),
            pltpu.SemaphoreType.DMA,
        ))
    def seg(srcp, dstp, fstack, accs_out, acc, src_v, dst_v, rows_v, zbuf,
            sem):
        c = lax.axis_index("c")
        s = lax.axis_index("s")
        coff = c * n

        _fill_zero(zbuf, 64, h)

        @pl.loop(0, rpt // 64)
        def _(i):
            pltpu.sync_copy(zbuf, acc.at[pl.ds(s * rpt + i * 64, 64)])

        plsc.subcore_barrier()

        @pl.loop(0, nch)
        def _(i):
            base = s * ept + i * CH
            pltpu.sync_copy(srcp.at[pl.ds(base, CH)], src_v)
            pltpu.sync_copy(dstp.at[pl.ds(base, CH)], dst_v)
            for j in range(CH // LANES):
                sl = pl.ds(j * LANES, LANES)
                src_v[sl] = src_v[sl] + coff
            pltpu.async_copy(fstack.at[src_v], rows_v, sem).wait()
            pltpu.sync_copy(rows_v, acc.at[dst_v], add=True)

        plsc.subcore_barrier()

        @pl.loop(0, rpt // 64)
        def _(i):
            r0 = s * rpt + i * 64
            pltpu.sync_copy(acc.at[pl.ds(r0, 64)], zbuf)
            pltpu.sync_copy(zbuf, accs_out.at[pl.ds(c * npad + r0, 64)])

    return seg


def _make_deg(n, h, npad, ept):
    rpt = npad // NS
    nch = ept // CH
    nch2 = nch // 2
    assert rpt % 64 == 0 and nch * CH == ept and nch2 * 2 == nch

    @functools.partial(
        pl.kernel,
        out_type=jax.ShapeDtypeStruct((2 * npad, h), jnp.float32),
        mesh=_sc_mesh(),
        scratch_types=(
            pltpu.VMEM_SHARED((npad, h), jnp.float32),
            pltpu.VMEM((CH,), jnp.int32),
            pltpu.VMEM((CH, h), jnp.float32),
            pltpu.VMEM((64, h), jnp.float32),
            pltpu.SemaphoreType.DMA,
        ))
    def degk(dstp, deg_out, dacc, dst_v, ones_v, zbuf, sem):
        del sem
        c = lax.axis_index("c")
        s = lax.axis_index("s")

        _fill_zero(zbuf, 64, h)
        one = jnp.ones((LANES,), jnp.float32)
        for r in range(CH):
            for j in range(h // LANES):
                ones_v[r, pl.ds(j * LANES, LANES)] = one

        @pl.loop(0, rpt // 64)
        def _(i):
            pltpu.sync_copy(zbuf, dacc.at[pl.ds(s * rpt + i * 64, 64)])

        plsc.subcore_barrier()

        @pl.loop(0, nch2)
        def _(i):
            base = s * ept + (c * nch2 + i) * CH
            pltpu.sync_copy(dstp.at[pl.ds(base, CH)], dst_v)
            pltpu.sync_copy(ones_v, dacc.at[dst_v], add=True)

        plsc.subcore_barrier()

        @pl.loop(0, rpt // 64)
        def _(i):
            r0 = s * rpt + i * 64
            pltpu.sync_copy(dacc.at[pl.ds(r0, 64)], zbuf)
            pltpu.sync_copy(zbuf, deg_out.at[pl.ds(c * npad + r0, 64)])

    return degk


def _tc_layer_body(lo_ref, hi_ref, f_ref, da_ref, db_ref, w_ref, b_ref,
                   g_ref, be_ref, out_ref, olo_ref, ohi_ref):
    h = lo_ref.shape[1]
    x = jnp.concatenate([lo_ref[...], hi_ref[...]], axis=1)
    deg = da_ref[:, 0:1] + db_ref[:, 0:1]
    agg = (x + f_ref[...]) / (deg + 1.0)
    y = jnp.dot(agg, w_ref[...], preferred_element_type=jnp.float32) + b_ref[...]
    mu = jnp.mean(y, axis=-1, keepdims=True)
    var = jnp.mean((y - mu) * (y - mu), axis=-1, keepdims=True)
    z = (y - mu) * lax.rsqrt(var + 1e-5) * g_ref[...] + be_ref[...]
    e = jnp.where(z > 0, z, jnp.exp(z) - 1.0)
    out_ref[...] = e
    olo_ref[...] = e[:, :h]
    ohi_ref[...] = e[:, h:]


def _tc_layer(acc_lo, acc_hi, feats, deg_a, deg_b, w, b, g, be, rows):
    n, d = feats.shape
    h = d // 2
    grid = n // rows
    blk = lambda r, cdim: pl.BlockSpec((r, cdim), lambda i: (i, 0))
    fixed = lambda shape: pl.BlockSpec(shape, lambda i: (0, 0))
    return pl.pallas_call(
        _tc_layer_body,
        grid=(grid,),
        in_specs=[blk(rows, h), blk(rows, h), blk(rows, d), blk(rows, h),
                  blk(rows, h), fixed((d, d)), fixed((1, d)), fixed((1, d)),
                  fixed((1, d))],
        out_specs=[blk(rows, d), blk(rows, h), blk(rows, h)],
        out_shape=[jax.ShapeDtypeStruct((n, d), jnp.float32),
                   jax.ShapeDtypeStruct((n, h), jnp.float32),
                   jax.ShapeDtypeStruct((n, h), jnp.float32)],
    )(acc_lo, acc_hi, feats, deg_a, deg_b, w, b, g, be)


def _make_row_gather(n, d, b):
    nw = NC * NS
    bw = b // nw
    assert bw % 8 == 0

    @functools.partial(
        pl.kernel, out_type=jax.ShapeDtypeStruct((b, d), jnp.float32),
        mesh=_sc_mesh(),
        scratch_types=(pltpu.VMEM((bw,), jnp.int32),
                       pltpu.VMEM((bw, d), jnp.float32),
                       pltpu.SemaphoreType.DMA))
    def gat(table, idx, out, idx_v, rows_v, sem):
        wid = lax.axis_index("s") * NC + lax.axis_index("c")
        base = wid * bw
        pltpu.sync_copy(idx.at[pl.ds(base, bw)], idx_v)
        pltpu.async_copy(table.at[idx_v], rows_v, sem).wait()
        pltpu.sync_copy(rows_v, out.at[pl.ds(base, bw)])

    return gat


def kernel(index, edge_index, embedding, W, b, gamma, beta):
    n, d = embedding.shape
    h = d // 2
    e = edge_index.shape[1]
    bsz = index.shape[0]

    ept = ((e + NS * 2 * CH - 1) // (NS * 2 * CH)) * 2 * CH
    epad = NS * ept
    rpt = ((n + 1 + NS * 64 - 1) // (NS * 64)) * 64
    npad = NS * rpt

    src = edge_index[0]
    dst = edge_index[1]
    pad = epad - e
    srcp = jnp.concatenate([src, jnp.zeros((pad,), jnp.int32)])
    dstp = jnp.concatenate([dst, jnp.full((pad,), n, jnp.int32)])

    fstack0 = jnp.concatenate([embedding[:, :h], embedding[:, h:]], axis=0)

    seg = _make_segsum(n, h, npad, ept)
    gat = _make_row_gather(n, d, bsz)

    accs1 = seg(srcp, dstp, fstack0)
    deg2 = _make_deg(n, h, npad, ept)(dstp)
    deg_a = deg2[:n]
    deg_b = deg2[npad:npad + n]
    f1, f1lo, f1hi = _tc_layer(accs1[:n], accs1[npad:npad + n], embedding,
                               deg_a, deg_b, W[0], b[0][None, :],
                               gamma[0][None, :], beta[0][None, :], rows=400)
    fstack1 = jnp.concatenate([f1lo, f1hi], axis=0)
    accs2 = seg(srcp, dstp, fstack1)
    f2, _, _ = _tc_layer(accs2[:n], accs2[npad:npad + n], f1, deg_a, deg_b,
                         W[1], b[1][None, :], gamma[1][None, :],
                         beta[1][None, :], rows=400)
    return gat(f2, index)


# speedup vs baseline: 1.2068x; 1.2068x over previous
import functools

import jax
import jax.numpy as jnp
from jax import lax
from jax.experimental import pallas as pl
from jax.experimental.pallas import tpu as pltpu
from jax.experimental.pallas import tpu_sc as plsc

NC = 2
NS = 16
LANES = 16
CH = 128


def _sc_mesh():
    return plsc.VectorSubcoreMesh(
        core_axis_name="c", subcore_axis_name="s", num_cores=NC, num_subcores=NS)


def _fill_zero(ref, nrows, ncols):
    z = jnp.zeros((LANES,), jnp.float32)
    for r in range(nrows):
        for j in range(ncols // LANES):
            ref[r, pl.ds(j * LANES, LANES)] = z


def _make_segsum(n, h, npad, ept):
    rpt = npad // NS
    nch = ept // CH
    assert rpt % 8 == 0 and nch * CH == ept

    @functools.partial(
        pl.kernel,
        out_type=jax.ShapeDtypeStruct((2 * npad, h), jnp.float32),
        mesh=_sc_mesh(),
        scratch_types=(
            pltpu.VMEM_SHARED((npad, h), jnp.float32),
            pltpu.VMEM((CH,), jnp.int32),
            pltpu.VMEM((CH,), jnp.int32),
            pltpu.VMEM((CH, h), jnp.float32),
            pltpu.VMEM((8, h), jnp.float32),
            pltpu.SemaphoreType.DMA,
        ))
    def seg(srcp, dstp, fstack, accs_out, acc, src_v, dst_v, rows_v, zbuf,
            sem):
        c = lax.axis_index("c")
        s = lax.axis_index("s")
        coff = c * n

        _fill_zero(zbuf, 8, h)

        @pl.loop(0, rpt // 8)
        def _(i):
            pltpu.sync_copy(zbuf, acc.at[pl.ds(s * rpt + i * 8, 8)])

        plsc.subcore_barrier()

        @pl.loop(0, nch)
        def _(i):
            base = s * ept + i * CH
            pltpu.sync_copy(srcp.at[pl.ds(base, CH)], src_v)
            pltpu.sync_copy(dstp.at[pl.ds(base, CH)], dst_v)
            for j in range(CH // LANES):
                sl = pl.ds(j * LANES, LANES)
                src_v[sl] = src_v[sl] + coff
            pltpu.async_copy(fstack.at[src_v], rows_v, sem).wait()
            pltpu.sync_copy(rows_v, acc.at[dst_v], add=True)

        plsc.subcore_barrier()

        @pl.loop(0, rpt // 8)
        def _(i):
            r0 = s * rpt + i * 8
            pltpu.sync_copy(acc.at[pl.ds(r0, 8)], zbuf)
            pltpu.sync_copy(zbuf, accs_out.at[pl.ds(c * npad + r0, 8)])

    return seg


def _make_deg(n, h, npad, ept):
    rpt = npad // NS
    nch = ept // CH
    assert rpt % 8 == 0 and nch * CH == ept

    @functools.partial(
        pl.kernel,
        out_type=jax.ShapeDtypeStruct((2 * npad, h), jnp.float32),
        mesh=_sc_mesh(),
        scratch_types=(
            pltpu.VMEM_SHARED((npad, h), jnp.float32),
            pltpu.VMEM((CH,), jnp.int32),
            pltpu.VMEM((CH, h), jnp.float32),
            pltpu.VMEM((8, h), jnp.float32),
            pltpu.SemaphoreType.DMA,
        ))
    def degk(dstp, deg_out, dacc, dst_v, ones_v, zbuf, sem):
        del sem
        c = lax.axis_index("c")
        s = lax.axis_index("s")

        _fill_zero(zbuf, 8, h)
        one = jnp.ones((LANES,), jnp.float32)
        for r in range(CH):
            for j in range(h // LANES):
                ones_v[r, pl.ds(j * LANES, LANES)] = one

        @pl.loop(0, rpt // 8)
        def _(i):
            pltpu.sync_copy(zbuf, dacc.at[pl.ds(s * rpt + i * 8, 8)])

        plsc.subcore_barrier()

        @pl.loop(0, nch)
        def _(i):
            base = s * ept + i * CH
            pltpu.sync_copy(dstp.at[pl.ds(base, CH)], dst_v)
            pltpu.sync_copy(ones_v, dacc.at[dst_v], add=True)

        plsc.subcore_barrier()

        @pl.loop(0, rpt // 8)
        def _(i):
            r0 = s * rpt + i * 8
            pltpu.sync_copy(dacc.at[pl.ds(r0, 8)], zbuf)
            pltpu.sync_copy(zbuf, deg_out.at[pl.ds(c * npad + r0, 8)])

    return degk


def _tc_layer_body(lo_ref, hi_ref, f_ref, d_ref, w_ref, b_ref, g_ref, be_ref,
                   out_ref, olo_ref, ohi_ref):
    h = lo_ref.shape[1]
    x = jnp.concatenate([lo_ref[...], hi_ref[...]], axis=1)
    deg = d_ref[:, 0:1]
    agg = (x + f_ref[...]) / (deg + 1.0)
    y = jnp.dot(agg, w_ref[...], preferred_element_type=jnp.float32) + b_ref[...]
    mu = jnp.mean(y, axis=-1, keepdims=True)
    var = jnp.mean((y - mu) * (y - mu), axis=-1, keepdims=True)
    z = (y - mu) * lax.rsqrt(var + 1e-5) * g_ref[...] + be_ref[...]
    e = jnp.where(z > 0, z, jnp.exp(z) - 1.0)
    out_ref[...] = e
    olo_ref[...] = e[:, :h]
    ohi_ref[...] = e[:, h:]


def _tc_layer(acc_lo, acc_hi, feats, deg16, w, b, g, be, rows):
    n, d = feats.shape
    h = d // 2
    grid = n // rows
    blk = lambda r, cdim: pl.BlockSpec((r, cdim), lambda i: (i, 0))
    fixed = lambda shape: pl.BlockSpec(shape, lambda i: (0, 0))
    return pl.pallas_call(
        _tc_layer_body,
        grid=(grid,),
        in_specs=[blk(rows, h), blk(rows, h), blk(rows, d), blk(rows, h),
                  fixed((d, d)), fixed((1, d)), fixed((1, d)), fixed((1, d))],
        out_specs=[blk(rows, d), blk(rows, h), blk(rows, h)],
        out_shape=[jax.ShapeDtypeStruct((n, d), jnp.float32),
                   jax.ShapeDtypeStruct((n, h), jnp.float32),
                   jax.ShapeDtypeStruct((n, h), jnp.float32)],
    )(acc_lo, acc_hi, feats, deg16, w, b, g, be)


def _make_row_gather(n, d, b):
    nw = NC * NS
    bw = b // nw
    assert bw % 8 == 0

    @functools.partial(
        pl.kernel, out_type=jax.ShapeDtypeStruct((b, d), jnp.float32),
        mesh=_sc_mesh(),
        scratch_types=(pltpu.VMEM((bw,), jnp.int32),
                       pltpu.VMEM((bw, d), jnp.float32),
                       pltpu.SemaphoreType.DMA))
    def gat(table, idx, out, idx_v, rows_v, sem):
        wid = lax.axis_index("s") * NC + lax.axis_index("c")
        base = wid * bw
        pltpu.sync_copy(idx.at[pl.ds(base, bw)], idx_v)
        pltpu.async_copy(table.at[idx_v], rows_v, sem).wait()
        pltpu.sync_copy(rows_v, out.at[pl.ds(base, bw)])

    return gat


def kernel(index, edge_index, embedding, W, b, gamma, beta):
    n, d = embedding.shape
    h = d // 2
    e = edge_index.shape[1]
    bsz = index.shape[0]

    ept = ((e + NS * CH - 1) // (NS * CH)) * CH
    epad = NS * ept
    rpt = ((n + 1 + NS * 8 - 1) // (NS * 8)) * 8
    npad = NS * rpt

    src = edge_index[0]
    dst = edge_index[1]
    pad = epad - e
    srcp = jnp.concatenate([src, jnp.zeros((pad,), jnp.int32)])
    dstp = jnp.concatenate([dst, jnp.full((pad,), n, jnp.int32)])

    fstack0 = jnp.concatenate([embedding[:, :h], embedding[:, h:]], axis=0)

    seg = _make_segsum(n, h, npad, ept)
    gat = _make_row_gather(n, d, bsz)

    accs1 = seg(srcp, dstp, fstack0)
    deg128 = _make_deg(n, h, npad, ept)(dstp)[:n]
    f1, f1lo, f1hi = _tc_layer(accs1[:n], accs1[npad:npad + n], embedding,
                               deg128, W[0], b[0][None, :], gamma[0][None, :],
                               beta[0][None, :], rows=400)
    fstack1 = jnp.concatenate([f1lo, f1hi], axis=0)
    accs2 = seg(srcp, dstp, fstack1)
    f2, _, _ = _tc_layer(accs2[:n], accs2[npad:npad + n], f1, deg128,
                         W[1], b[1][None, :], gamma[1][None, :],
                         beta[1][None, :], rows=400)
    return gat(f2, index)
